# baseline (speedup 1.0000x reference)
import functools

import jax
import jax.numpy as jnp
from jax import lax
from jax.experimental import pallas as pl
from jax.experimental.pallas import tpu as pltpu
from jax.experimental.pallas import tpu_sc as plsc

N = 10000
E = 320000
D = 128
H = 8
DH = 16

NC = 2
NS = 16
NW = NC * NS

C = 48
CHUNKS = 210
EPW = C * CHUNKS
EPAD = EPW * NW
EALLOC = EPAD + C
NPAD = 10112
ROWS_PER_TILE = NPAD // NS

_SCALE = float(D) ** -0.5
_EPS = 1e-5


def _pre_body(x_ref, wqk_ref, wv_ref, qk_ref, v_ref):
    x = x_ref[...]
    qk_ref[:, :D] = jnp.dot(x, wqk_ref[:, :D],
                            preferred_element_type=jnp.float32) * _SCALE
    qk_ref[:, D:] = jnp.dot(x, wqk_ref[:, D:],
                            preferred_element_type=jnp.float32)
    v_ref[...] = jnp.dot(x, wv_ref[...], preferred_element_type=jnp.float32)


def _dense_pre(x, W_qk, W_v):
    blk = 1000
    grid = N // blk
    return pl.pallas_call(
        _pre_body,
        grid=(grid,),
        in_specs=[
            pl.BlockSpec((blk, D), lambda i: (i, 0)),
            pl.BlockSpec((D, 2 * D), lambda i: (0, 0)),
            pl.BlockSpec((D, D), lambda i: (0, 0)),
        ],
        out_specs=[
            pl.BlockSpec((blk, 2 * D), lambda i: (i, 0)),
            pl.BlockSpec((blk, D), lambda i: (i, 0)),
        ],
        out_shape=[
            jax.ShapeDtypeStruct((N, 2 * D), jnp.float32),
            jax.ShapeDtypeStruct((N, D), jnp.float32),
        ],
    )(x, W_qk, W_v)


def _sc_body(c0_hbm, c1_hbm, v_hbm, sd_hbm, out_hbm,
             sda, sdb,
             c0a, c1a, va, c0b, c1b, vb, zbuf, acc, sem):
    cid = lax.axis_index("c")
    sid = lax.axis_index("s")
    wid = cid * NS + sid

    zero16 = jnp.zeros((16,), jnp.float32)
    for r in range(8):
        for l in range(D // 16):
            zbuf[r, pl.ds(16 * l, 16)] = zero16
    base = sid * ROWS_PER_TILE

    def zchunk(k, carry):
        o = pl.multiple_of(base + 8 * k, 8)
        pltpu.sync_copy(zbuf, acc.at[pl.ds(o, 8)])
        return carry

    lax.fori_loop(0, ROWS_PER_TILE // 8, zchunk, 0)
    plsc.subcore_barrier()

    ebase = wid * EPW

    lanes = lax.iota(jnp.int32, 16)
    _dn = lax.GatherDimensionNumbers(
        offset_dims=(), collapsed_slice_dims=(0,), start_index_map=(0,))

    def permx(t, k):
        idx = (lanes ^ k)[:, None]
        return lax.gather(t, idx, dimension_numbers=_dn, slice_sizes=(1,),
                          mode=lax.GatherScatterMode.PROMISE_IN_BOUNDS)

    masks = {k: (lanes & k) == 0 for k in (1, 2, 4, 8)}

    cbase = wid * CHUNKS

    def load_idx(j, sd):
        pltpu.sync_copy(sd_hbm.at[cbase + j], sd)

    def fire(sd, c0r, c1r, vr):
        pltpu.async_copy(c0_hbm.at[sd.at[0]], c0r, sem)
        pltpu.async_copy(c1_hbm.at[sd.at[1]], c1r, sem)
        pltpu.async_copy(v_hbm.at[sd.at[0]], vr, sem)

    def drain(sd, c0r, c1r, vr):
        pltpu.make_async_copy(c0_hbm.at[sd.at[0]], c0r, sem).wait()
        pltpu.make_async_copy(c1_hbm.at[sd.at[1]], c1r, sem).wait()
        pltpu.make_async_copy(v_hbm.at[sd.at[0]], vr, sem).wait()

    def tree(vals, op):
        while len(vals) > 1:
            vals = [op(vals[2 * a], vals[2 * a + 1])
                    for a in range(len(vals) // 2)]
        return vals[0]

    def bcast(t, e):
        idx = jnp.full((16, 1), e, jnp.int32)
        return lax.gather(t, idx, dimension_numbers=_dn, slice_sizes=(1,),
                          mode=lax.GatherScatterMode.PROMISE_IN_BOUNDS)

    def combine(a, b, k):
        return jnp.where(masks[k], a + permx(a, k), b + permx(b, k))

    def compute_scatter(dv, c0r, c1r, vr):
        @plsc.parallel_loop(0, C // 16, unroll=3)
        def _group(g):
            s = []
            for h in range(H):
                sl = pl.ds(16 * h, 16)
                vs = [c1r[g * 16 + e, sl] * c0r[g * 16 + e, sl]
                      for e in range(16)]
                for k in (8, 4, 2, 1):
                    vs = [combine(vs[a], vs[a + len(vs) // 2], k)
                          for a in range(len(vs) // 2)]
                s.append(vs[0])
            m = tree(list(s), jnp.maximum)
            evs = [jnp.exp(s[h] - m) for h in range(H)]
            den = tree(list(evs), lambda a, b: a + b)
            r = 1.0 / den
            ps = [evs[h] * r for h in range(H)]
            for e in range(16):
                i = g * 16 + e
                for h in range(H):
                    ph = bcast(ps[h], e)
                    vr[i, pl.ds(16 * h, 16)] = vr[i, pl.ds(16 * h, 16)] * ph

        pltpu.sync_copy(vr, acc.at[dv], add=True)

    load_idx(0, sda)
    fire(sda, c0a, c1a, va)

    def pair_body(k, carry):
        j0 = 2 * k
        load_idx(j0 + 1, sdb)
        fire(sdb, c0b, c1b, vb)
        drain(sda, c0a, c1a, va)
        compute_scatter(sda.at[1], c0a, c1a, va)
        load_idx(j0 + 2, sda)
        fire(sda, c0a, c1a, va)
        drain(sdb, c0b, c1b, vb)
        compute_scatter(sdb.at[1], c0b, c1b, vb)
        return carry

    lax.fori_loop(0, CHUNKS // 2, pair_body, 0)
    drain(sda, c0a, c1a, va)
    plsc.subcore_barrier()

    pltpu.sync_copy(acc.at[pl.ds(base, ROWS_PER_TILE)],
                    out_hbm.at[cid, pl.ds(base, ROWS_PER_TILE)])


def _sc_message_passing(c0p, c1p, vp, sdp):
    mesh = plsc.VectorSubcoreMesh(core_axis_name="c", subcore_axis_name="s")
    kfun = pl.kernel(
        _sc_body,
        out_type=jax.ShapeDtypeStruct((NC, NPAD, D), jnp.float32),
        mesh=mesh,
        scratch_types=[
            pltpu.VMEM((2, C), jnp.int32),
            pltpu.VMEM((2, C), jnp.int32),
            pltpu.VMEM((C, D), jnp.float32),
            pltpu.VMEM((C, D), jnp.float32),
            pltpu.VMEM((C, D), jnp.float32),
            pltpu.VMEM((C, D), jnp.float32),
            pltpu.VMEM((C, D), jnp.float32),
            pltpu.VMEM((C, D), jnp.float32),
            pltpu.VMEM((8, D), jnp.float32),
            pltpu.VMEM_SHARED((NPAD, D), jnp.float32),
            pltpu.SemaphoreType.DMA,
        ],
    )
    return kfun(c0p, c1p, vp, sdp)


def _ln(z, g, b):
    mu = jnp.mean(z, axis=-1, keepdims=True)
    var = jnp.mean((z - mu) ** 2, axis=-1, keepdims=True)
    return (z - mu) / jnp.sqrt(var + _EPS) * g + b


def _post_body(p0_ref, p1_ref, x_ref, wo_ref, bo_ref, w1_ref, b1_ref,
               w2_ref, b2_ref, g_ref, be_ref, o_ref):
    agg = p0_ref[...] + p1_ref[...]
    x = x_ref[...]
    g = g_ref[...]
    be = be_ref[...]
    attn_out = jnp.dot(agg, wo_ref[...],
                       preferred_element_type=jnp.float32) + bo_ref[...]
    h = _ln(attn_out + x, g, be)
    h = jax.nn.relu(jnp.dot(h, w1_ref[...],
                            preferred_element_type=jnp.float32) + b1_ref[...])
    h = jnp.dot(h, w2_ref[...], preferred_element_type=jnp.float32) + b2_ref[...]
    o_ref[...] = jax.nn.relu(_ln(h, g, be))


def _dense_post(p0, p1, x, W_out, b_out, W1, b1, W2, b2, ln_g, ln_b):
    blk = 1000
    grid = N // blk
    row = lambda i: (i, 0)
    fix = lambda i: (0, 0)
    return pl.pallas_call(
        _post_body,
        grid=(grid,),
        in_specs=[
            pl.BlockSpec((blk, D), row),
            pl.BlockSpec((blk, D), row),
            pl.BlockSpec((blk, D), row),
            pl.BlockSpec((D, D), fix),
            pl.BlockSpec((1, D), fix),
            pl.BlockSpec((D, D), fix),
            pl.BlockSpec((1, D), fix),
            pl.BlockSpec((D, D), fix),
            pl.BlockSpec((1, D), fix),
            pl.BlockSpec((1, D), fix),
            pl.BlockSpec((1, D), fix),
        ],
        out_specs=pl.BlockSpec((blk, D), row),
        out_shape=jax.ShapeDtypeStruct((N, D), jnp.float32),
    )(p0, p1, x, W_out, b_out.reshape(1, D), W1, b1.reshape(1, D),
      W2, b2.reshape(1, D), ln_g.reshape(1, D), ln_b.reshape(1, D))


def kernel(x, edge_index, W_qk, W_v, W_out, b_out, W1, b1, W2, b2, ln_g, ln_b):
    qk, v = _dense_pre(x, W_qk, W_v)
    c0 = qk[:, :D]
    c1 = qk[:, D:]

    padn = jnp.zeros((NPAD - N, D), jnp.float32)
    c0p = jnp.concatenate([c0, padn], axis=0)
    c1p = jnp.concatenate([c1, padn], axis=0)
    vp = jnp.concatenate([v, padn], axis=0)

    src = edge_index[0]
    dst = edge_index[1]
    pade = jnp.zeros((EALLOC - E,), jnp.int32)
    srcp = jnp.concatenate([src, pade])
    junk = N + jnp.arange(EALLOC - E, dtype=jnp.int32) % (NPAD - N)
    dstp = jnp.concatenate([dst, junk])
    sdp = jnp.stack([srcp.reshape(-1, C), dstp.reshape(-1, C)], axis=1)

    parts = _sc_message_passing(c0p, c1p, vp, sdp)

    return _dense_post(parts[0, :N], parts[1, :N], x,
                       W_out, b_out, W1, b1, W2, b2, ln_g, ln_b)


# speedup vs baseline: 1.7815x; 1.7815x over previous
import functools

import jax
import jax.numpy as jnp
from jax import lax
from jax.experimental import pallas as pl
from jax.experimental.pallas import tpu as pltpu
from jax.experimental.pallas import tpu_sc as plsc

N = 10000
E = 320000
D = 128
H = 8
DH = 16

NC = 2
NS = 16
NW = NC * NS

C = 48
CHUNKS = 210
EPW = C * CHUNKS
EPAD = EPW * NW
EALLOC = EPAD + C
NPAD = 10112
ROWS_PER_TILE = NPAD // NS

_SCALE = float(D) ** -0.5
_EPS = 1e-5


def _pre_body(x_ref, wqk_ref, wv_ref, qk_ref, v_ref):
    x = x_ref[...]
    qk_ref[:, :D] = jnp.dot(x, wqk_ref[:, :D],
                            preferred_element_type=jnp.float32) * _SCALE
    qk_ref[:, D:] = jnp.dot(x, wqk_ref[:, D:],
                            preferred_element_type=jnp.float32)
    v_ref[...] = jnp.dot(x, wv_ref[...], preferred_element_type=jnp.float32)


def _dense_pre(x, W_qk, W_v):
    blk = 1000
    grid = N // blk
    return pl.pallas_call(
        _pre_body,
        grid=(grid,),
        in_specs=[
            pl.BlockSpec((blk, D), lambda i: (i, 0)),
            pl.BlockSpec((D, 2 * D), lambda i: (0, 0)),
            pl.BlockSpec((D, D), lambda i: (0, 0)),
        ],
        out_specs=[
            pl.BlockSpec((blk, 2 * D), lambda i: (i, 0)),
            pl.BlockSpec((blk, D), lambda i: (i, 0)),
        ],
        out_shape=[
            jax.ShapeDtypeStruct((N, 2 * D), jnp.float32),
            jax.ShapeDtypeStruct((N, D), jnp.float32),
        ],
    )(x, W_qk, W_v)


def _sc_body(c0_hbm, c1_hbm, v_hbm, sd_hbm, out_hbm,
             sda, sdb,
             c0a, c1a, va, c0b, c1b, vb, zbuf, acc, sem):
    cid = lax.axis_index("c")
    sid = lax.axis_index("s")
    wid = cid * NS + sid

    zero16 = jnp.zeros((16,), jnp.float32)
    for r in range(8):
        for l in range(D // 16):
            zbuf[r, pl.ds(16 * l, 16)] = zero16
    base = sid * ROWS_PER_TILE

    def zchunk(k, carry):
        o = pl.multiple_of(base + 8 * k, 8)
        pltpu.sync_copy(zbuf, acc.at[pl.ds(o, 8)])
        return carry

    lax.fori_loop(0, ROWS_PER_TILE // 8, zchunk, 0)
    plsc.subcore_barrier()

    ebase = wid * EPW

    lanes = lax.iota(jnp.int32, 16)
    _dn = lax.GatherDimensionNumbers(
        offset_dims=(), collapsed_slice_dims=(0,), start_index_map=(0,))

    def permx(t, k):
        idx = (lanes ^ k)[:, None]
        return lax.gather(t, idx, dimension_numbers=_dn, slice_sizes=(1,),
                          mode=lax.GatherScatterMode.PROMISE_IN_BOUNDS)

    masks = {k: (lanes & k) == 0 for k in (1, 2, 4, 8)}

    cbase = wid * CHUNKS

    def load_idx(j, sd):
        pltpu.sync_copy(sd_hbm.at[cbase + j], sd)

    def fire(sd, c0r, c1r, vr):
        pltpu.async_copy(c0_hbm.at[sd.at[0]], c0r, sem)
        pltpu.async_copy(c1_hbm.at[sd.at[1]], c1r, sem)
        pltpu.async_copy(v_hbm.at[sd.at[0]], vr, sem)

    def drain(sd, c0r, c1r, vr):
        pltpu.make_async_copy(c0_hbm.at[sd.at[0]], c0r, sem).wait()
        pltpu.make_async_copy(c1_hbm.at[sd.at[1]], c1r, sem).wait()
        pltpu.make_async_copy(v_hbm.at[sd.at[0]], vr, sem).wait()

    def tree(vals, op):
        while len(vals) > 1:
            vals = [op(vals[2 * a], vals[2 * a + 1])
                    for a in range(len(vals) // 2)]
        return vals[0]

    def bcast(t, e):
        idx = jnp.full((16, 1), e, jnp.int32)
        return lax.gather(t, idx, dimension_numbers=_dn, slice_sizes=(1,),
                          mode=lax.GatherScatterMode.PROMISE_IN_BOUNDS)

    def combine(a, b, k):
        return jnp.where(masks[k], a + permx(a, k), b + permx(b, k))

    def compute_scatter(dv, c0r, c1r, vr):
        @plsc.parallel_loop(0, C // 16, unroll=1)
        def _group(g):
            s = []
            for h in range(H):
                sl = pl.ds(16 * h, 16)
                vs = []
                for e in range(8):
                    a = c1r[g * 16 + e, sl] * c0r[g * 16 + e, sl]
                    b = c1r[g * 16 + e + 8, sl] * c0r[g * 16 + e + 8, sl]
                    vs.append(combine(a, b, 8))
                for k in (4, 2, 1):
                    vs = [combine(vs[a], vs[a + len(vs) // 2], k)
                          for a in range(len(vs) // 2)]
                s.append(vs[0])
            m = tree(list(s), jnp.maximum)
            evs = [jnp.exp(s[h] - m) for h in range(H)]
            den = tree(list(evs), lambda a, b: a + b)
            r = 1.0 / den
            ps = [evs[h] * r for h in range(H)]
            for e in range(16):
                i = g * 16 + e
                for h in range(H):
                    ph = bcast(ps[h], e)
                    vr[i, pl.ds(16 * h, 16)] = vr[i, pl.ds(16 * h, 16)] * ph

        pltpu.sync_copy(vr, acc.at[dv], add=True)

    load_idx(0, sda)
    fire(sda, c0a, c1a, va)

    def pair_body(k, carry):
        j0 = 2 * k
        load_idx(j0 + 1, sdb)
        fire(sdb, c0b, c1b, vb)
        drain(sda, c0a, c1a, va)
        compute_scatter(sda.at[1], c0a, c1a, va)
        load_idx(j0 + 2, sda)
        fire(sda, c0a, c1a, va)
        drain(sdb, c0b, c1b, vb)
        compute_scatter(sdb.at[1], c0b, c1b, vb)
        return carry

    lax.fori_loop(0, CHUNKS // 2, pair_body, 0)
    drain(sda, c0a, c1a, va)
    plsc.subcore_barrier()

    pltpu.sync_copy(acc.at[pl.ds(base, ROWS_PER_TILE)],
                    out_hbm.at[cid, pl.ds(base, ROWS_PER_TILE)])


def _sc_message_passing(c0p, c1p, vp, sdp):
    mesh = plsc.VectorSubcoreMesh(core_axis_name="c", subcore_axis_name="s")
    kfun = pl.kernel(
        _sc_body,
        out_type=jax.ShapeDtypeStruct((NC, NPAD, D), jnp.float32),
        mesh=mesh,
        scratch_types=[
            pltpu.VMEM((2, C), jnp.int32),
            pltpu.VMEM((2, C), jnp.int32),
            pltpu.VMEM((C, D), jnp.float32),
            pltpu.VMEM((C, D), jnp.float32),
            pltpu.VMEM((C, D), jnp.float32),
            pltpu.VMEM((C, D), jnp.float32),
            pltpu.VMEM((C, D), jnp.float32),
            pltpu.VMEM((C, D), jnp.float32),
            pltpu.VMEM((8, D), jnp.float32),
            pltpu.VMEM_SHARED((NPAD, D), jnp.float32),
            pltpu.SemaphoreType.DMA,
        ],
    )
    return kfun(c0p, c1p, vp, sdp)


def _ln(z, g, b):
    mu = jnp.mean(z, axis=-1, keepdims=True)
    var = jnp.mean((z - mu) ** 2, axis=-1, keepdims=True)
    return (z - mu) / jnp.sqrt(var + _EPS) * g + b


def _post_body(p0_ref, p1_ref, x_ref, wo_ref, bo_ref, w1_ref, b1_ref,
               w2_ref, b2_ref, g_ref, be_ref, o_ref):
    agg = p0_ref[...] + p1_ref[...]
    x = x_ref[...]
    g = g_ref[...]
    be = be_ref[...]
    attn_out = jnp.dot(agg, wo_ref[...],
                       preferred_element_type=jnp.float32) + bo_ref[...]
    h = _ln(attn_out + x, g, be)
    h = jax.nn.relu(jnp.dot(h, w1_ref[...],
                            preferred_element_type=jnp.float32) + b1_ref[...])
    h = jnp.dot(h, w2_ref[...], preferred_element_type=jnp.float32) + b2_ref[...]
    o_ref[...] = jax.nn.relu(_ln(h, g, be))


def _dense_post(p0, p1, x, W_out, b_out, W1, b1, W2, b2, ln_g, ln_b):
    blk = 1000
    grid = N // blk
    row = lambda i: (i, 0)
    fix = lambda i: (0, 0)
    return pl.pallas_call(
        _post_body,
        grid=(grid,),
        in_specs=[
            pl.BlockSpec((blk, D), row),
            pl.BlockSpec((blk, D), row),
            pl.BlockSpec((blk, D), row),
            pl.BlockSpec((D, D), fix),
            pl.BlockSpec((1, D), fix),
            pl.BlockSpec((D, D), fix),
            pl.BlockSpec((1, D), fix),
            pl.BlockSpec((D, D), fix),
            pl.BlockSpec((1, D), fix),
            pl.BlockSpec((1, D), fix),
            pl.BlockSpec((1, D), fix),
        ],
        out_specs=pl.BlockSpec((blk, D), row),
        out_shape=jax.ShapeDtypeStruct((N, D), jnp.float32),
    )(p0, p1, x, W_out, b_out.reshape(1, D), W1, b1.reshape(1, D),
      W2, b2.reshape(1, D), ln_g.reshape(1, D), ln_b.reshape(1, D))


def kernel(x, edge_index, W_qk, W_v, W_out, b_out, W1, b1, W2, b2, ln_g, ln_b):
    qk, v = _dense_pre(x, W_qk, W_v)
    c0 = qk[:, :D]
    c1 = qk[:, D:]

    padn = jnp.zeros((NPAD - N, D), jnp.float32)
    c0p = jnp.concatenate([c0, padn], axis=0)
    c1p = jnp.concatenate([c1, padn], axis=0)
    vp = jnp.concatenate([v, padn], axis=0)

    src = edge_index[0]
    dst = edge_index[1]
    pade = jnp.zeros((EALLOC - E,), jnp.int32)
    srcp = jnp.concatenate([src, pade])
    junk = N + jnp.arange(EALLOC - E, dtype=jnp.int32) % (NPAD - N)
    dstp = jnp.concatenate([dst, junk])
    sdp = jnp.stack([srcp.reshape(-1, C), dstp.reshape(-1, C)], axis=1)

    parts = _sc_message_passing(c0p, c1p, vp, sdp)

    return _dense_post(parts[0, :N], parts[1, :N], x,
                       W_out, b_out, W1, b1, W2, b2, ln_g, ln_b)
